# baseline (speedup 1.0000x reference)
import jax
import jax.numpy as jnp
from jax.experimental import pallas as pl
from jax.experimental.pallas import tpu as pltpu
from jax.experimental.pallas import tpu_sc as plsc

N = 4096
D = 1024
F = 64
E = 8
H = 2048
O = 1024
BANDS = 4
R = 16
ALPHA = 32.0
CAPF = 1.25
CAP = int(CAPF * N / E)
SCALE = ALPHA / R
BR = BANDS * R
_SC_INTERPRET = False


def _mlp_body(xe_ref, snr_ref, band_ref, g_ref,
              w1_ref, b1_ref, w2_ref, b2_ref,
              a1_ref, bl1_ref, a2_ref, bl2_ref, yw_ref):
    xe = xe_ref[0]
    snr_col = snr_ref[0]
    band_col = band_ref[0]
    g_col = g_ref[0]

    xef = jnp.concatenate([xe, snr_col], axis=1)
    xb = xef.astype(jnp.bfloat16)

    colband = jax.lax.broadcasted_iota(jnp.int32, (CAP, BR), 1) // R
    mask = (colband == band_col).astype(jnp.float32)

    h = jnp.dot(xb, w1_ref[0].astype(jnp.bfloat16),
                preferred_element_type=jnp.float32)
    h += b1_ref[0]
    t1 = jnp.dot(xb, a1_ref[0].astype(jnp.bfloat16),
                 preferred_element_type=jnp.float32)
    h += SCALE * jnp.dot((t1 * mask).astype(jnp.bfloat16),
                         bl1_ref[0].astype(jnp.bfloat16),
                         preferred_element_type=jnp.float32)
    h = jax.nn.gelu(h)
    hb = h.astype(jnp.bfloat16)

    y = jnp.dot(hb, w2_ref[0].astype(jnp.bfloat16),
                preferred_element_type=jnp.float32)
    y += b2_ref[0]
    t2 = jnp.dot(hb, a2_ref[0].astype(jnp.bfloat16),
                 preferred_element_type=jnp.float32)
    y += SCALE * jnp.dot((t2 * mask).astype(jnp.bfloat16),
                         bl2_ref[0].astype(jnp.bfloat16),
                         preferred_element_type=jnp.float32)

    yw_ref[0] = y * g_col


def _combine_body(yw_ref, idx_ref, scoresT_ref, out_ref, aux_ref, imp_ref):
    e = pl.program_id(0)
    idx_row = idx_ref[0]

    tok = jax.lax.broadcasted_iota(jnp.int32, (N, CAP), 0)
    onehot = (tok == idx_row).astype(jnp.bfloat16)

    @pl.when(e == 0)
    def _():
        out_ref[...] = jnp.zeros(out_ref.shape, out_ref.dtype)

    out_ref[...] += jnp.dot(onehot, yw_ref[0].astype(jnp.bfloat16),
                            preferred_element_type=jnp.float32)

    imp = jnp.sum(scoresT_ref[0])
    imp_ref[pl.ds(e, 1), :] = jnp.full((1, 128), imp, jnp.float32)

    @pl.when(e == E - 1)
    def _():
        col = imp_ref[:, 0:1]
        m = jnp.mean(col)
        var = jnp.mean((col - m) ** 2)
        aux_ref[...] = jnp.full((1, 1), var / (m * m + 1e-10), jnp.float32)


def _bs(shape):
    return pl.BlockSpec((1,) + shape, lambda e: (e,) + (0,) * len(shape))


def _mlp(xe, snr_sel, band_sel, g, W1, b1, W2, b2, A1f, B1f, A2f, B2f):
    return pl.pallas_call(
        _mlp_body,
        grid=(E,),
        in_specs=[
            _bs((CAP, D)),
            _bs((CAP, 1)),
            _bs((CAP, 1)),
            _bs((CAP, 1)),
            _bs((D + 1, H)),
            _bs((1, H)),
            _bs((H, O)),
            _bs((1, O)),
            _bs((D + 1, BR)),
            _bs((BR, H)),
            _bs((H, BR)),
            _bs((BR, O)),
        ],
        out_specs=_bs((CAP, O)),
        out_shape=jax.ShapeDtypeStruct((E, CAP, O), jnp.float32),
    )(xe, snr_sel, band_sel, g, W1, b1, W2, b2, A1f, B1f, A2f, B2f)


def _combine(yw, idx, scoresT):
    out, aux = pl.pallas_call(
        _combine_body,
        grid=(E,),
        in_specs=[
            _bs((CAP, O)),
            _bs((1, CAP)),
            _bs((1, N)),
        ],
        out_specs=[
            pl.BlockSpec((N, O), lambda e: (0, 0)),
            pl.BlockSpec((1, 1), lambda e: (0, 0)),
        ],
        out_shape=[
            jax.ShapeDtypeStruct((N, O), jnp.float32),
            jax.ShapeDtypeStruct((1, 1), jnp.float32),
        ],
        scratch_shapes=[pltpu.VMEM((E, 128), jnp.float32)],
    )(yw, idx, scoresT)
    return out, aux[0, 0]


NSUB = 16
NQ = 4
QTOK = N // NQ
QV = QTOK // 16
SLOTS = E * CAP
WSLOT = CAP // NQ
GCH = 32
NCH = WSLOT // GCH
ESC = E // 2


def _sc_route_body(scoresT, scoresT_bits, x_hbm, snr_hbm, band_hbm,
                   idx_out, g_out, xe_out, snr_out, band_out, dbg_out,
                   sc_vm, bits_vm, snr_tile, band_tile, dbg_vm,
                   tok2d, gsc2d, pos2d, cnt_vm, cnt2_vm, tmp4_vm, tmp4b_vm,
                   myidx2d, rows_vm, snrsel_vm, bandsel_vm,
                   shared_cnt, shared_gt, shared_eq,
                   shared_idx, shared_g, sem):
    c = jax.lax.axis_index("c")
    s = jax.lax.axis_index("s")
    e = c * NQ + s // NQ
    q = s % NQ
    wid = c * NSUB + s
    qv16 = jnp.full((16,), q, jnp.int32)
    capv = jnp.full((16,), CAP, jnp.int32)
    zero16 = jnp.zeros((16,), jnp.int32)

    pltpu.sync_copy(scoresT.at[e, pl.ds(q * QTOK, QTOK)], sc_vm)
    pltpu.sync_copy(scoresT_bits.at[e], bits_vm)
    pltpu.sync_copy(snr_hbm, snr_tile)
    pltpu.sync_copy(band_hbm, band_tile)

    one16 = jnp.full((16,), 1, jnp.int32)
    NV = N // 16

    def count_gt_full(pivot):
        def body(k, acc):
            m = bits_vm[pl.ds(k * 16, 16)] > pivot
            return acc + jnp.where(m, one16, zero16)
        return jax.lax.fori_loop(0, NV, body, zero16)

    splat = lambda v: jnp.full((16,), jnp.sum(v), jnp.int32)

    def bs_body(_, lohi):
        lo, hi = lohi
        mid = (lo + hi) >> 1
        total = splat(count_gt_full(mid))
        ge = total >= capv
        return (jnp.where(ge, mid + 1, lo), jnp.where(ge, hi, mid))

    lo0 = jnp.zeros((16,), jnp.int32)
    hi0 = jnp.full((16,), 0x7F800000, jnp.int32)
    _, vthr = jax.lax.fori_loop(0, 31, bs_body, (lo0, hi0))

    qbase = q * QTOK

    def count_quarter(j):
        def body(k, acc):
            gtc, eqc = acc
            b = bits_vm[pl.ds(j * QTOK + k * 16, 16)]
            gtc = gtc + jnp.where(b > vthr, one16, zero16)
            eqc = eqc + jnp.where(b == vthr, one16, zero16)
            return (gtc, eqc)
        gl, el_ = jax.lax.fori_loop(0, QV, body, (zero16, zero16))
        return splat(gl), splat(el_)

    g0, e0 = count_quarter(0)
    g1, e1 = count_quarter(1)
    g2, e2 = count_quarter(2)
    g3, e3 = count_quarter(3)

    n_gt = g0 + g1 + g2 + g3
    m_eq = capv - n_gt
    p0, p1, p2, p3 = zero16, e0, e0 + e1, e0 + e1 + e2
    clip = lambda t, emax: jnp.minimum(jnp.maximum(t, zero16), emax)
    t0 = clip(m_eq - p0, e0)
    t1 = clip(m_eq - p1, e1)
    t2 = clip(m_eq - p2, e2)
    t3 = clip(m_eq - p3, e3)
    sel_q = lambda a0, a1, a2, a3: jnp.where(
        qv16 == 0, a0, jnp.where(qv16 == 1, a1,
                                 jnp.where(qv16 == 2, a2, a3)))
    my_take = sel_q(t0, t1, t2, t3)
    off01 = g0 + t0
    off02 = off01 + g1 + t1
    off03 = off02 + g2 + t2
    my_off = sel_q(zero16, off01, off02, off03)
    my_cnt = sel_q(g0, g1, g2, g3) + my_take

    lane = jax.lax.iota(jnp.int32, 16)

    def walk(k, carry):
        eqc, outc = carry
        b = bits_vm[pl.ds(qbase + k * 16, 16)]
        sc = sc_vm[pl.ds(k * 16, 16)]
        gt = b > vthr
        eq = b == vthr
        cs_eq = plsc.cumsum(jnp.where(eq, one16, zero16))
        eq_rank = eqc + cs_eq
        sel = jnp.logical_or(gt, jnp.logical_and(eq, eq_rank <= my_take))
        cs_sel = plsc.cumsum(jnp.where(sel, one16, zero16))
        pos = jnp.clip(outc + cs_sel - 1, 0, CAP - 1)
        tok = lane + (k * 16 + q * QTOK)
        plsc.store_scatter(tok2d, [(pos >> 7) & 7, pos & 127], tok, mask=sel)
        plsc.store_scatter(gsc2d, [(pos >> 7) & 7, pos & 127], sc, mask=sel)
        return (eqc + jnp.full((16,), cs_eq[15], jnp.int32),
                outc + jnp.full((16,), cs_sel[15], jnp.int32))
    jax.lax.fori_loop(0, QV, walk, (zero16, zero16))

    el = e % ESC
    lbase = el * CAP + q * WSLOT
    for t in range(WSLOT // 16):
        bandsel_vm[pl.ds(t * 16, 16)] = zero16
    pltpu.sync_copy(bandsel_vm, shared_idx.at[pl.ds(lbase, WSLOT)])
    plsc.subcore_barrier()

    base = jnp.full((16,), el * CAP, jnp.int32) + my_off
    padbase = jnp.full((16,), ESC * CAP + s * CAP, jnp.int32)
    for r in range(NCH):
        for cc in range(8):
            i_vec = lane + (r * 128 + cc * 16)
            dest = jnp.where(i_vec < my_cnt, base + i_vec, padbase + i_vec)
            pos2d[r, pl.ds(cc * 16, 16)] = jnp.clip(
                dest, 0, (ESC + NSUB) * CAP - 1)

    for r in range(NCH):
        pltpu.sync_copy(tok2d.at[r], shared_idx.at[pos2d.at[r]])
        pltpu.sync_copy(gsc2d.at[r], shared_g.at[pos2d.at[r]])
    plsc.subcore_barrier()

    dbg_vm[pl.ds(0, 16)] = vthr
    dbg_vm[pl.ds(16, 16)] = sel_q(g0, g1, g2, g3)
    dbg_vm[pl.ds(32, 16)] = sel_q(e0, e1, e2, e3)
    dbg_vm[pl.ds(48, 16)] = my_off
    dbg_vm[pl.ds(64, 16)] = my_cnt
    dbg_vm[pl.ds(80, 16)] = my_take
    pltpu.sync_copy(dbg_vm, dbg_out.at[wid])

    slot0 = e * CAP + q * WSLOT
    for r in range(NCH):
        pltpu.sync_copy(shared_idx.at[pl.ds(lbase + r * GCH, GCH)],
                        myidx2d.at[r])
        pltpu.sync_copy(myidx2d.at[r],
                        idx_out.at[pl.ds(slot0 + r * GCH, GCH)])
    pltpu.sync_copy(shared_g.at[pl.ds(lbase, WSLOT)], snrsel_vm)
    pltpu.sync_copy(snrsel_vm, g_out.at[pl.ds(slot0, WSLOT)])


def _sc_route(scoresT, x, snr_flat, band_ids):
    scoresT_bits = jax.lax.bitcast_convert_type(scoresT, jnp.int32)
    mesh = plsc.VectorSubcoreMesh(core_axis_name="c", subcore_axis_name="s")
    fn = pl.kernel(
        _sc_route_body,
        mesh=mesh,
        compiler_params=pltpu.CompilerParams(needs_layout_passes=False),
        interpret=_SC_INTERPRET,
        out_type=[
            jax.ShapeDtypeStruct((SLOTS,), jnp.int32),
            jax.ShapeDtypeStruct((SLOTS,), jnp.float32),
            jax.ShapeDtypeStruct((SLOTS, D), jnp.float32),
            jax.ShapeDtypeStruct((SLOTS,), jnp.float32),
            jax.ShapeDtypeStruct((SLOTS,), jnp.int32),
            jax.ShapeDtypeStruct((32, 96), jnp.int32),
        ],
        scratch_types=[
            pltpu.VMEM((QTOK,), jnp.float32),
            pltpu.VMEM((N,), jnp.int32),
            pltpu.VMEM((N,), jnp.float32),
            pltpu.VMEM((N,), jnp.int32),
            pltpu.VMEM((96,), jnp.int32),
            pltpu.VMEM((8, 128), jnp.int32),
            pltpu.VMEM((8, 128), jnp.float32),
            pltpu.VMEM((NCH, 128), jnp.int32),
            pltpu.VMEM((16,), jnp.int32),
            pltpu.VMEM((16,), jnp.int32),
            pltpu.VMEM((NQ, 16), jnp.int32),
            pltpu.VMEM((NQ, 16), jnp.int32),
            pltpu.VMEM((NCH, GCH), jnp.int32),
            pltpu.VMEM((GCH, D), jnp.float32),
            pltpu.VMEM((WSLOT,), jnp.float32),
            pltpu.VMEM((WSLOT,), jnp.int32),
            pltpu.VMEM_SHARED((NSUB, 16), jnp.int32),
            pltpu.VMEM_SHARED((NSUB, 16), jnp.int32),
            pltpu.VMEM_SHARED((NSUB, 16), jnp.int32),
            pltpu.VMEM_SHARED(((ESC + NSUB) * CAP,), jnp.int32),
            pltpu.VMEM_SHARED(((ESC + NSUB) * CAP,), jnp.float32),
            pltpu.SemaphoreType.DMA,
        ],
    )
    return fn(scoresT, scoresT_bits, x, snr_flat, band_ids)


def kernel(x, snr, band_ids, W_dct, W_gate, W1, b1, W2, b2, A1, B1, A2, B2):
    freq = x @ W_dct
    logits = jnp.concatenate([x, freq], axis=-1) @ W_gate
    scores = jax.nn.softmax(logits, axis=-1)

    (idx_pad, g_pad, xe_flat, snr_flat_sel, band_flat_sel,
     _sc_dbg) = _sc_route(scores.T, x, snr[:, 0], band_ids)
    idx = idx_pad.reshape(E, CAP)
    g = g_pad.reshape(E, CAP)
    xe = x[idx]
    snr_sel = snr[:, 0][idx][..., None]
    band_sel = band_ids[idx][..., None]

    A1f = jnp.transpose(A1, (0, 2, 1, 3)).reshape(E, D + 1, BR)
    B1f = B1.reshape(E, BR, H)
    A2f = jnp.transpose(A2, (0, 2, 1, 3)).reshape(E, H, BR)
    B2f = B2.reshape(E, BR, O)

    yw = _mlp(xe, snr_sel, band_sel, g[..., None],
              W1, b1.reshape(E, 1, H), W2, b2.reshape(E, 1, O),
              A1f, B1f, A2f, B2f)
    out, aux_loss = _combine(yw, idx[:, None, :], scores.T[:, None, :])
    return out, aux_loss


# speedup vs baseline: 1.0073x; 1.0073x over previous
import jax
import jax.numpy as jnp
from jax.experimental import pallas as pl
from jax.experimental.pallas import tpu as pltpu
from jax.experimental.pallas import tpu_sc as plsc

N = 4096
D = 1024
F = 64
E = 8
H = 2048
O = 1024
BANDS = 4
R = 16
ALPHA = 32.0
CAPF = 1.25
CAP = int(CAPF * N / E)
SCALE = ALPHA / R
BR = BANDS * R


def _mlp_body(xe_ref, snr_ref, band_ref, g_ref,
              w1_ref, b1_ref, w2_ref, b2_ref,
              a1_ref, bl1_ref, a2_ref, bl2_ref, yw_ref):
    xe = xe_ref[0]
    snr_col = snr_ref[0]
    band_col = band_ref[0]
    g_col = g_ref[0]

    xef = jnp.concatenate([xe, snr_col], axis=1)
    xb = xef.astype(jnp.bfloat16)

    colband = jax.lax.broadcasted_iota(jnp.int32, (CAP, BR), 1) // R
    mask = (colband == band_col).astype(jnp.float32)

    h = jnp.dot(xb, w1_ref[0].astype(jnp.bfloat16),
                preferred_element_type=jnp.float32)
    h += b1_ref[0]
    t1 = jnp.dot(xb, a1_ref[0].astype(jnp.bfloat16),
                 preferred_element_type=jnp.float32)
    h += SCALE * jnp.dot((t1 * mask).astype(jnp.bfloat16),
                         bl1_ref[0].astype(jnp.bfloat16),
                         preferred_element_type=jnp.float32)
    h = jax.nn.gelu(h)
    hb = h.astype(jnp.bfloat16)

    y = jnp.dot(hb, w2_ref[0].astype(jnp.bfloat16),
                preferred_element_type=jnp.float32)
    y += b2_ref[0]
    t2 = jnp.dot(hb, a2_ref[0].astype(jnp.bfloat16),
                 preferred_element_type=jnp.float32)
    y += SCALE * jnp.dot((t2 * mask).astype(jnp.bfloat16),
                         bl2_ref[0].astype(jnp.bfloat16),
                         preferred_element_type=jnp.float32)

    yw_ref[0] = y * g_col


def _combine_body(yw_ref, idx_ref, scoresT_ref, out_ref, aux_ref, imp_ref):
    e = pl.program_id(0)
    idx_row = idx_ref[0]

    tok = jax.lax.broadcasted_iota(jnp.int32, (N, CAP), 0)
    onehot = (tok == idx_row).astype(jnp.bfloat16)

    @pl.when(e == 0)
    def _():
        out_ref[...] = jnp.zeros(out_ref.shape, out_ref.dtype)

    out_ref[...] += jnp.dot(onehot, yw_ref[0].astype(jnp.bfloat16),
                            preferred_element_type=jnp.float32)

    imp = jnp.sum(scoresT_ref[0])
    imp_ref[pl.ds(e, 1), :] = jnp.full((1, 128), imp, jnp.float32)

    @pl.when(e == E - 1)
    def _():
        col = imp_ref[:, 0:1]
        m = jnp.mean(col)
        var = jnp.mean((col - m) ** 2)
        aux_ref[...] = jnp.full((1, 1), var / (m * m + 1e-10), jnp.float32)


def _bs(shape):
    return pl.BlockSpec((1,) + shape, lambda e: (e,) + (0,) * len(shape))


def _mlp(xe, snr_sel, band_sel, g, W1, b1, W2, b2, A1f, B1f, A2f, B2f):
    return pl.pallas_call(
        _mlp_body,
        grid=(E,),
        in_specs=[
            _bs((CAP, D)),
            _bs((CAP, 1)),
            _bs((CAP, 1)),
            _bs((CAP, 1)),
            _bs((D + 1, H)),
            _bs((1, H)),
            _bs((H, O)),
            _bs((1, O)),
            _bs((D + 1, BR)),
            _bs((BR, H)),
            _bs((H, BR)),
            _bs((BR, O)),
        ],
        out_specs=_bs((CAP, O)),
        out_shape=jax.ShapeDtypeStruct((E, CAP, O), jnp.float32),
    )(xe, snr_sel, band_sel, g, W1, b1, W2, b2, A1f, B1f, A2f, B2f)


def _combine(yw, idx, scoresT):
    out, aux = pl.pallas_call(
        _combine_body,
        grid=(E,),
        in_specs=[
            _bs((CAP, O)),
            _bs((1, CAP)),
            _bs((1, N)),
        ],
        out_specs=[
            pl.BlockSpec((N, O), lambda e: (0, 0)),
            pl.BlockSpec((1, 1), lambda e: (0, 0)),
        ],
        out_shape=[
            jax.ShapeDtypeStruct((N, O), jnp.float32),
            jax.ShapeDtypeStruct((1, 1), jnp.float32),
        ],
        scratch_shapes=[pltpu.VMEM((E, 128), jnp.float32)],
    )(yw, idx, scoresT)
    return out, aux[0, 0]


NSUB = 16
NQ = 4
QTOK = N // NQ
QV = QTOK // 16
SLOTS = E * CAP
WSLOT = CAP // NQ
GCH = 32
NCH = WSLOT // GCH
ESC = E // 2


def _sc_route_body(scoresT, scoresT_bits,
                   idx_out, g_out, dbg_out,
                   sc_vm, bits_vm, dbg_vm,
                   tok2d, gsc2d, pos2d,
                   myidx2d, snrsel_vm, bandsel_vm,
                   shared_idx, shared_g, sem):
    c = jax.lax.axis_index("c")
    s = jax.lax.axis_index("s")
    e = c * NQ + s // NQ
    q = s % NQ
    wid = c * NSUB + s
    qv16 = jnp.full((16,), q, jnp.int32)
    capv = jnp.full((16,), CAP, jnp.int32)
    zero16 = jnp.zeros((16,), jnp.int32)

    pltpu.sync_copy(scoresT.at[e, pl.ds(q * QTOK, QTOK)], sc_vm)
    pltpu.sync_copy(scoresT_bits.at[e], bits_vm)

    one16 = jnp.full((16,), 1, jnp.int32)
    NV = N // 16

    def count_gt_full(pivot):
        def body(k, acc):
            m = bits_vm[pl.ds(k * 16, 16)] > pivot
            return acc + jnp.where(m, one16, zero16)
        return jax.lax.fori_loop(0, NV, body, zero16)

    splat = lambda v: jnp.full((16,), jnp.sum(v), jnp.int32)

    def bs_body(_, lohi):
        lo, hi = lohi
        mid = (lo + hi) >> 1
        total = splat(count_gt_full(mid))
        ge = total >= capv
        return (jnp.where(ge, mid + 1, lo), jnp.where(ge, hi, mid))

    lo0 = jnp.zeros((16,), jnp.int32)
    hi0 = jnp.full((16,), 0x7F800000, jnp.int32)
    _, vthr = jax.lax.fori_loop(0, 31, bs_body, (lo0, hi0))

    qbase = q * QTOK

    def count_quarter(j):
        def body(k, acc):
            gtc, eqc = acc
            b = bits_vm[pl.ds(j * QTOK + k * 16, 16)]
            gtc = gtc + jnp.where(b > vthr, one16, zero16)
            eqc = eqc + jnp.where(b == vthr, one16, zero16)
            return (gtc, eqc)
        gl, el_ = jax.lax.fori_loop(0, QV, body, (zero16, zero16))
        return splat(gl), splat(el_)

    g0, e0 = count_quarter(0)
    g1, e1 = count_quarter(1)
    g2, e2 = count_quarter(2)
    g3, e3 = count_quarter(3)

    n_gt = g0 + g1 + g2 + g3
    m_eq = capv - n_gt
    p0, p1, p2, p3 = zero16, e0, e0 + e1, e0 + e1 + e2
    clip = lambda t, emax: jnp.minimum(jnp.maximum(t, zero16), emax)
    t0 = clip(m_eq - p0, e0)
    t1 = clip(m_eq - p1, e1)
    t2 = clip(m_eq - p2, e2)
    t3 = clip(m_eq - p3, e3)
    sel_q = lambda a0, a1, a2, a3: jnp.where(
        qv16 == 0, a0, jnp.where(qv16 == 1, a1,
                                 jnp.where(qv16 == 2, a2, a3)))
    my_take = sel_q(t0, t1, t2, t3)
    off01 = g0 + t0
    off02 = off01 + g1 + t1
    off03 = off02 + g2 + t2
    my_off = sel_q(zero16, off01, off02, off03)
    my_cnt = sel_q(g0, g1, g2, g3) + my_take

    lane = jax.lax.iota(jnp.int32, 16)

    def walk(k, carry):
        eqc, outc = carry
        b = bits_vm[pl.ds(qbase + k * 16, 16)]
        sc = sc_vm[pl.ds(k * 16, 16)]
        gt = b > vthr
        eq = b == vthr
        cs_eq = plsc.cumsum(jnp.where(eq, one16, zero16))
        eq_rank = eqc + cs_eq
        sel = jnp.logical_or(gt, jnp.logical_and(eq, eq_rank <= my_take))
        cs_sel = plsc.cumsum(jnp.where(sel, one16, zero16))
        pos = jnp.clip(outc + cs_sel - 1, 0, CAP - 1)
        tok = lane + (k * 16 + q * QTOK)
        plsc.store_scatter(tok2d, [(pos >> 7) & 7, pos & 127], tok, mask=sel)
        plsc.store_scatter(gsc2d, [(pos >> 7) & 7, pos & 127], sc, mask=sel)
        return (eqc + jnp.full((16,), cs_eq[15], jnp.int32),
                outc + jnp.full((16,), cs_sel[15], jnp.int32))
    jax.lax.fori_loop(0, QV, walk, (zero16, zero16))

    el = e % ESC
    lbase = el * CAP + q * WSLOT
    for t in range(WSLOT // 16):
        bandsel_vm[pl.ds(t * 16, 16)] = zero16
    pltpu.sync_copy(bandsel_vm, shared_idx.at[pl.ds(lbase, WSLOT)])
    plsc.subcore_barrier()

    base = jnp.full((16,), el * CAP, jnp.int32) + my_off
    padbase = jnp.full((16,), ESC * CAP + s * CAP, jnp.int32)
    for r in range(NCH):
        for cc in range(8):
            i_vec = lane + (r * 128 + cc * 16)
            dest = jnp.where(i_vec < my_cnt, base + i_vec, padbase + i_vec)
            pos2d[r, pl.ds(cc * 16, 16)] = jnp.clip(
                dest, 0, (ESC + NSUB) * CAP - 1)

    for r in range(NCH):
        pltpu.sync_copy(tok2d.at[r], shared_idx.at[pos2d.at[r]])
        pltpu.sync_copy(gsc2d.at[r], shared_g.at[pos2d.at[r]])
    plsc.subcore_barrier()

    dbg_vm[pl.ds(0, 16)] = vthr
    dbg_vm[pl.ds(16, 16)] = sel_q(g0, g1, g2, g3)
    dbg_vm[pl.ds(32, 16)] = sel_q(e0, e1, e2, e3)
    dbg_vm[pl.ds(48, 16)] = my_off
    dbg_vm[pl.ds(64, 16)] = my_cnt
    dbg_vm[pl.ds(80, 16)] = my_take
    pltpu.sync_copy(dbg_vm, dbg_out.at[wid])

    slot0 = e * CAP + q * WSLOT
    for r in range(NCH):
        pltpu.sync_copy(shared_idx.at[pl.ds(lbase + r * GCH, GCH)],
                        myidx2d.at[r])
        pltpu.sync_copy(myidx2d.at[r],
                        idx_out.at[pl.ds(slot0 + r * GCH, GCH)])
    pltpu.sync_copy(shared_g.at[pl.ds(lbase, WSLOT)], snrsel_vm)
    pltpu.sync_copy(snrsel_vm, g_out.at[pl.ds(slot0, WSLOT)])


def _sc_route(scoresT):
    scoresT_bits = jax.lax.bitcast_convert_type(scoresT, jnp.int32)
    mesh = plsc.VectorSubcoreMesh(core_axis_name="c", subcore_axis_name="s")
    fn = pl.kernel(
        _sc_route_body,
        mesh=mesh,
        compiler_params=pltpu.CompilerParams(needs_layout_passes=False),
        out_type=[
            jax.ShapeDtypeStruct((SLOTS,), jnp.int32),
            jax.ShapeDtypeStruct((SLOTS,), jnp.float32),
            jax.ShapeDtypeStruct((32, 96), jnp.int32),
        ],
        scratch_types=[
            pltpu.VMEM((QTOK,), jnp.float32),
            pltpu.VMEM((N,), jnp.int32),
            pltpu.VMEM((96,), jnp.int32),
            pltpu.VMEM((8, 128), jnp.int32),
            pltpu.VMEM((8, 128), jnp.float32),
            pltpu.VMEM((NCH, 128), jnp.int32),
            pltpu.VMEM((NCH, GCH), jnp.int32),
            pltpu.VMEM((WSLOT,), jnp.float32),
            pltpu.VMEM((WSLOT,), jnp.int32),
            pltpu.VMEM_SHARED(((ESC + NSUB) * CAP,), jnp.int32),
            pltpu.VMEM_SHARED(((ESC + NSUB) * CAP,), jnp.float32),
            pltpu.SemaphoreType.DMA,
        ],
    )
    return fn(scoresT, scoresT_bits)


def kernel(x, snr, band_ids, W_dct, W_gate, W1, b1, W2, b2, A1, B1, A2, B2):
    freq = x @ W_dct
    logits = jnp.concatenate([x, freq], axis=-1) @ W_gate
    scores = jax.nn.softmax(logits, axis=-1)

    idx_pad, g_pad, _sc_dbg = _sc_route(scores.T)
    idx = idx_pad.reshape(E, CAP)
    g = g_pad.reshape(E, CAP)
    xe = x[idx]
    snr_sel = snr[:, 0][idx][..., None]
    band_sel = band_ids[idx][..., None]

    A1f = jnp.transpose(A1, (0, 2, 1, 3)).reshape(E, D + 1, BR)
    B1f = B1.reshape(E, BR, H)
    A2f = jnp.transpose(A2, (0, 2, 1, 3)).reshape(E, H, BR)
    B2f = B2.reshape(E, BR, O)

    yw = _mlp(xe, snr_sel, band_sel, g[..., None],
              W1, b1.reshape(E, 1, H), W2, b2.reshape(E, 1, O),
              A1f, B1f, A2f, B2f)
    out, aux_loss = _combine(yw, idx[:, None, :], scores.T[:, None, :])
    return out, aux_loss
